# baseline (speedup 1.0000x reference)
import jax
import jax.numpy as jnp
from jax import lax
from jax.experimental import pallas as pl
from jax.experimental.pallas import tpu as pltpu

N = 16384
C = 1000
NB = 16
W = N // NB
S = 128
SEARCH_ITERS = 31


CHUNK = 1024
NCHUNK = N // CHUNK
DEPTH = 4


def _phase_a_kernel(x_hbm, lbl_ref, conf_ref, acc_ref, ring, sems):
    i = pl.program_id(0)
    slot = lax.rem(i, DEPTH)

    @pl.when(i == 0)
    def _prologue():
        for d in range(DEPTH):
            pltpu.make_async_copy(
                x_hbm.at[pl.ds(d * CHUNK, CHUNK), :],
                ring.at[d], sems.at[d]).start()

    pltpu.make_async_copy(
        x_hbm.at[pl.ds(i * CHUNK, CHUNK), :],
        ring.at[slot], sems.at[slot]).wait()

    x = ring[slot]
    m = jnp.max(x, axis=1, keepdims=True)
    col = lax.broadcasted_iota(jnp.int32, x.shape, 1)
    big = jnp.int32(2 ** 30)
    pidx = jnp.min(jnp.where(x == m, col, big), axis=1, keepdims=True)
    base = pl.multiple_of(i * CHUNK, CHUNK)
    conf_ref[pl.ds(base, CHUNK), :] = m
    acc_ref[pl.ds(base, CHUNK), :] = (pidx == lbl_ref[...]).astype(
        jnp.float32)

    @pl.when(i + DEPTH < NCHUNK)
    def _refill():
        pltpu.make_async_copy(
            x_hbm.at[pl.ds((i + DEPTH) * CHUNK, CHUNK), :],
            ring.at[slot], sems.at[slot]).start()


def _sum11(x):
    return jnp.sum(jnp.sum(x, axis=0, keepdims=True), axis=1, keepdims=True)


def _phase_b_kernel(conf_ref, acc_ref, ece_ref, ys_ref):
    conf = conf_ref[...]
    acc = acc_ref[...]
    u = lax.bitcast_convert_type(conf, jnp.int32)

    nb1 = NB - 1
    ranks_f = [jnp.full((1, 1), float((b + 1) * W), jnp.float32)
               for b in range(nb1)]

    def search_body(_, carry):
        los, his = carry
        nlos, nhis = [], []
        for b in range(nb1):
            lo, hi = los[b], his[b]
            mid = lo + lax.shift_right_arithmetic(hi - lo, jnp.int32(1))
            cnt = _sum11(jnp.where(u <= mid, 1.0, 0.0))
            pred = cnt >= ranks_f[b] + 1.0
            nlos.append(jnp.where(pred, lo, mid + 1))
            nhis.append(jnp.where(pred, mid, hi))
        return tuple(nlos), tuple(nhis)

    init = (tuple(jnp.zeros((1, 1), jnp.int32) for _ in range(nb1)),
            tuple(jnp.full((1, 1), 0x40000000, jnp.int32)
                  for _ in range(nb1)))
    vs, _ = lax.fori_loop(0, SEARCH_ITERS, search_body, init)

    r_iota = lax.broadcasted_iota(jnp.int32, (S, S), 0)
    c_iota = lax.broadcasted_iota(jnp.int32, (S, S), 1)
    l_strict = (c_iota < r_iota).astype(jnp.float32)
    u_strict = (r_iota < c_iota).astype(jnp.float32)
    ones_mat = jnp.ones((S, S), jnp.float32)
    binf = jnp.zeros((S, S), jnp.float32)
    for b in range(nb1):
        vb = vs[b]
        eq = (u == vb)
        nl = ranks_f[b] - _sum11(jnp.where(u < vb, 1.0, 0.0))
        mb = eq.astype(jnp.float32)
        t1 = jnp.dot(mb, u_strict, preferred_element_type=jnp.float32)
        rowtot = jnp.dot(mb, ones_mat, preferred_element_type=jnp.float32)
        t2 = jnp.dot(l_strict, rowtot, preferred_element_type=jnp.float32)
        tier = t1 + t2
        binf = (binf + jnp.where(u > vb, 1.0, 0.0)
                + jnp.where(eq & (tier >= nl), 1.0, 0.0))

    avgs, yss = [], []
    for k in range(NB):
        mk = (binf == float(k)).astype(jnp.float32)
        avgs.append(_sum11(conf * mk) * (1.0 / W))
        yss.append(_sum11(acc * mk) * (1.0 / W))
    avg_conf = jnp.concatenate(avgs, axis=1)
    ys_row = jnp.concatenate(yss, axis=1)
    ece = jnp.sum(jnp.abs(avg_conf - ys_row), axis=1,
                  keepdims=True) * (float(W) / float(N))
    ece_ref[...] = ece
    ys_ref[...] = ys_row


@jax.jit
def kernel(softmaxes, labels):
    lbl2 = labels.astype(jnp.int32).reshape(N, 1)
    conf, accv = pl.pallas_call(
        _phase_a_kernel,
        grid=(NCHUNK,),
        in_specs=[pl.BlockSpec(memory_space=pl.ANY),
                  pl.BlockSpec((CHUNK, 1), lambda i: (i, 0))],
        out_specs=[pl.BlockSpec((N, 1), lambda i: (0, 0)),
                   pl.BlockSpec((N, 1), lambda i: (0, 0))],
        out_shape=[jax.ShapeDtypeStruct((N, 1), jnp.float32),
                   jax.ShapeDtypeStruct((N, 1), jnp.float32)],
        scratch_shapes=[pltpu.VMEM((DEPTH, CHUNK, C), jnp.float32),
                        pltpu.SemaphoreType.DMA((DEPTH,))],
    )(softmaxes, lbl2)

    ece, ys = pl.pallas_call(
        _phase_b_kernel,
        in_specs=[pl.BlockSpec((S, S), lambda: (0, 0)),
                  pl.BlockSpec((S, S), lambda: (0, 0))],
        out_specs=[pl.BlockSpec((1, 1), lambda: (0, 0)),
                   pl.BlockSpec((1, NB), lambda: (0, 0))],
        out_shape=[jax.ShapeDtypeStruct((1, 1), jnp.float32),
                   jax.ShapeDtypeStruct((1, NB), jnp.float32)],
    )(conf.reshape(S, S), accv.reshape(S, S))
    return (ece.reshape(1), ys.reshape(NB))


# speedup vs baseline: 1.0392x; 1.0392x over previous
import jax
import jax.numpy as jnp
from jax import lax
from jax.experimental import pallas as pl
from jax.experimental.pallas import tpu as pltpu

N = 16384
C = 1000
NB = 16
W = N // NB
S = 128
SEARCH_ITERS = 31


BR = 4096
GRID_A = N // BR


def _phase_a_kernel(x_ref, lbl_ref, conf_ref, acc_ref):
    i = pl.program_id(0)
    base = pl.multiple_of(i * BR, BR)
    x = x_ref[...]
    m = jnp.max(x, axis=1, keepdims=True)
    col = lax.broadcasted_iota(jnp.int32, x.shape, 1)
    big = jnp.int32(2 ** 30)
    pidx = jnp.min(jnp.where(x == m, col, big), axis=1, keepdims=True)
    conf_ref[pl.ds(base, BR), :] = m
    acc_ref[pl.ds(base, BR), :] = (pidx == lbl_ref[...]).astype(jnp.float32)


def _sum11(x):
    return jnp.sum(jnp.sum(x, axis=0, keepdims=True), axis=1, keepdims=True)


def _phase_b_kernel(conf_ref, acc_ref, ece_ref, ys_ref):
    conf = conf_ref[...]
    acc = acc_ref[...]
    u = lax.bitcast_convert_type(conf, jnp.int32)

    nb1 = NB - 1
    ranks_f = [jnp.full((1, 1), float((b + 1) * W), jnp.float32)
               for b in range(nb1)]

    def search_body(_, carry):
        los, his = carry
        nlos, nhis = [], []
        for b in range(nb1):
            lo, hi = los[b], his[b]
            mid = lo + lax.shift_right_arithmetic(hi - lo, jnp.int32(1))
            cnt = _sum11(jnp.where(u <= mid, 1.0, 0.0))
            pred = cnt >= ranks_f[b] + 1.0
            nlos.append(jnp.where(pred, lo, mid + 1))
            nhis.append(jnp.where(pred, mid, hi))
        return tuple(nlos), tuple(nhis)

    init = (tuple(jnp.zeros((1, 1), jnp.int32) for _ in range(nb1)),
            tuple(jnp.full((1, 1), 0x40000000, jnp.int32)
                  for _ in range(nb1)))
    vs, _ = lax.fori_loop(0, SEARCH_ITERS, search_body, init)

    r_iota = lax.broadcasted_iota(jnp.int32, (S, S), 0)
    c_iota = lax.broadcasted_iota(jnp.int32, (S, S), 1)
    l_strict = (c_iota < r_iota).astype(jnp.float32)
    u_strict = (r_iota < c_iota).astype(jnp.float32)
    ones_mat = jnp.ones((S, S), jnp.float32)
    binf = jnp.zeros((S, S), jnp.float32)
    for b in range(nb1):
        vb = vs[b]
        eq = (u == vb)
        nl = ranks_f[b] - _sum11(jnp.where(u < vb, 1.0, 0.0))
        mb = eq.astype(jnp.float32)
        t1 = jnp.dot(mb, u_strict, preferred_element_type=jnp.float32)
        rowtot = jnp.dot(mb, ones_mat, preferred_element_type=jnp.float32)
        t2 = jnp.dot(l_strict, rowtot, preferred_element_type=jnp.float32)
        tier = t1 + t2
        binf = (binf + jnp.where(u > vb, 1.0, 0.0)
                + jnp.where(eq & (tier >= nl), 1.0, 0.0))

    avgs, yss = [], []
    for k in range(NB):
        mk = (binf == float(k)).astype(jnp.float32)
        avgs.append(_sum11(conf * mk) * (1.0 / W))
        yss.append(_sum11(acc * mk) * (1.0 / W))
    avg_conf = jnp.concatenate(avgs, axis=1)
    ys_row = jnp.concatenate(yss, axis=1)
    ece = jnp.sum(jnp.abs(avg_conf - ys_row), axis=1,
                  keepdims=True) * (float(W) / float(N))
    ece_ref[...] = ece
    ys_ref[...] = ys_row


@jax.jit
def kernel(softmaxes, labels):
    lbl2 = labels.astype(jnp.int32).reshape(N, 1)
    conf, accv = pl.pallas_call(
        _phase_a_kernel,
        grid=(GRID_A,),
        in_specs=[pl.BlockSpec((BR, C), lambda i: (i, 0)),
                  pl.BlockSpec((BR, 1), lambda i: (i, 0))],
        out_specs=[pl.BlockSpec((N, 1), lambda i: (0, 0)),
                   pl.BlockSpec((N, 1), lambda i: (0, 0))],
        out_shape=[jax.ShapeDtypeStruct((N, 1), jnp.float32),
                   jax.ShapeDtypeStruct((N, 1), jnp.float32)],
    )(softmaxes, lbl2)

    ece, ys = pl.pallas_call(
        _phase_b_kernel,
        in_specs=[pl.BlockSpec((S, S), lambda: (0, 0)),
                  pl.BlockSpec((S, S), lambda: (0, 0))],
        out_specs=[pl.BlockSpec((1, 1), lambda: (0, 0)),
                   pl.BlockSpec((1, NB), lambda: (0, 0))],
        out_shape=[jax.ShapeDtypeStruct((1, 1), jnp.float32),
                   jax.ShapeDtypeStruct((1, NB), jnp.float32)],
    )(conf.reshape(S, S), accv.reshape(S, S))
    return (ece.reshape(1), ys.reshape(NB))


# speedup vs baseline: 1.0451x; 1.0057x over previous
import jax
import jax.numpy as jnp
from jax import lax
from jax.experimental import pallas as pl
from jax.experimental.pallas import tpu as pltpu

N = 16384
C = 1000
NB = 16
W = N // NB
S = 128
SEARCH_ITERS = 31


BR = 2048
GRID_A = N // BR


def _phase_a_kernel(x_ref, lbl_ref, conf_ref, acc_ref):
    i = pl.program_id(0)
    base = pl.multiple_of(i * BR, BR)
    x = x_ref[...]
    m = jnp.max(x, axis=1, keepdims=True)
    col = lax.broadcasted_iota(jnp.int32, x.shape, 1)
    big = jnp.int32(2 ** 30)
    pidx = jnp.min(jnp.where(x == m, col, big), axis=1, keepdims=True)
    conf_ref[pl.ds(base, BR), :] = m
    acc_ref[pl.ds(base, BR), :] = (pidx == lbl_ref[...]).astype(jnp.float32)


def _sum11(x):
    return jnp.sum(jnp.sum(x, axis=0, keepdims=True), axis=1, keepdims=True)


def _phase_b_kernel(conf_ref, acc_ref, ece_ref, ys_ref):
    conf = conf_ref[...]
    acc = acc_ref[...]
    u = lax.bitcast_convert_type(conf, jnp.int32)

    nb1 = NB - 1
    ranks_f = [jnp.full((1, 1), float((b + 1) * W), jnp.float32)
               for b in range(nb1)]

    def search_body(_, carry):
        los, his = carry
        nlos, nhis = [], []
        for b in range(nb1):
            lo, hi = los[b], his[b]
            mid = lo + lax.shift_right_arithmetic(hi - lo, jnp.int32(1))
            cnt = _sum11(jnp.where(u <= mid, 1.0, 0.0))
            pred = cnt >= ranks_f[b] + 1.0
            nlos.append(jnp.where(pred, lo, mid + 1))
            nhis.append(jnp.where(pred, mid, hi))
        return tuple(nlos), tuple(nhis)

    init = (tuple(jnp.zeros((1, 1), jnp.int32) for _ in range(nb1)),
            tuple(jnp.full((1, 1), 0x40000000, jnp.int32)
                  for _ in range(nb1)))
    vs, _ = lax.fori_loop(0, SEARCH_ITERS, search_body, init)

    r_iota = lax.broadcasted_iota(jnp.int32, (S, S), 0)
    c_iota = lax.broadcasted_iota(jnp.int32, (S, S), 1)
    l_strict = (c_iota < r_iota).astype(jnp.float32)
    u_strict = (r_iota < c_iota).astype(jnp.float32)
    ones_mat = jnp.ones((S, S), jnp.float32)
    binf = jnp.zeros((S, S), jnp.float32)
    for b in range(nb1):
        vb = vs[b]
        eq = (u == vb)
        nl = ranks_f[b] - _sum11(jnp.where(u < vb, 1.0, 0.0))
        mb = eq.astype(jnp.float32)
        t1 = jnp.dot(mb, u_strict, preferred_element_type=jnp.float32)
        rowtot = jnp.dot(mb, ones_mat, preferred_element_type=jnp.float32)
        t2 = jnp.dot(l_strict, rowtot, preferred_element_type=jnp.float32)
        tier = t1 + t2
        binf = (binf + jnp.where(u > vb, 1.0, 0.0)
                + jnp.where(eq & (tier >= nl), 1.0, 0.0))

    avgs, yss = [], []
    for k in range(NB):
        mk = (binf == float(k)).astype(jnp.float32)
        avgs.append(_sum11(conf * mk) * (1.0 / W))
        yss.append(_sum11(acc * mk) * (1.0 / W))
    avg_conf = jnp.concatenate(avgs, axis=1)
    ys_row = jnp.concatenate(yss, axis=1)
    ece = jnp.sum(jnp.abs(avg_conf - ys_row), axis=1,
                  keepdims=True) * (float(W) / float(N))
    ece_ref[...] = ece
    ys_ref[...] = ys_row


@jax.jit
def kernel(softmaxes, labels):
    lbl2 = labels.astype(jnp.int32).reshape(N, 1)
    conf, accv = pl.pallas_call(
        _phase_a_kernel,
        grid=(GRID_A,),
        in_specs=[pl.BlockSpec((BR, C), lambda i: (i, 0)),
                  pl.BlockSpec((BR, 1), lambda i: (i, 0))],
        out_specs=[pl.BlockSpec((N, 1), lambda i: (0, 0)),
                   pl.BlockSpec((N, 1), lambda i: (0, 0))],
        out_shape=[jax.ShapeDtypeStruct((N, 1), jnp.float32),
                   jax.ShapeDtypeStruct((N, 1), jnp.float32)],
    )(softmaxes, lbl2)

    ece, ys = pl.pallas_call(
        _phase_b_kernel,
        in_specs=[pl.BlockSpec((S, S), lambda: (0, 0)),
                  pl.BlockSpec((S, S), lambda: (0, 0))],
        out_specs=[pl.BlockSpec((1, 1), lambda: (0, 0)),
                   pl.BlockSpec((1, NB), lambda: (0, 0))],
        out_shape=[jax.ShapeDtypeStruct((1, 1), jnp.float32),
                   jax.ShapeDtypeStruct((1, NB), jnp.float32)],
    )(conf.reshape(S, S), accv.reshape(S, S))
    return (ece.reshape(1), ys.reshape(NB))
